# Initial kernel scaffold; baseline (speedup 1.0000x reference)
import functools

import jax
import jax.numpy as jnp
from jax import lax
from jax.experimental import pallas as pl
from jax.experimental.pallas import tpu as pltpu
from jax.experimental.pallas import tpu_sc as plsc

N_NODES_C = 10000
N_EDGES_C = 320000
NC = 2
NS = 16
NW = NC * NS
EDGES_PER_W = N_EDGES_C // NW
BLK = 80
NBLK = EDGES_PER_W // BLK
ROWS_PER_TILE = N_NODES_C // NS

_MESH = plsc.VectorSubcoreMesh(core_axis_name="c", subcore_axis_name="s")


@functools.partial(
    pl.kernel,
    out_type=jax.ShapeDtypeStruct((NC, N_NODES_C), jnp.float32),
    mesh=_MESH,
    scratch_types=[
        pltpu.VMEM_SHARED((N_NODES_C,), jnp.float32),
        pltpu.VMEM((NBLK, BLK), jnp.int32),
        pltpu.VMEM((NBLK, BLK), jnp.float32),
    ],
)
def _sc_degree(dst_hbm, ew_hbm, zero_hbm, out_hbm, acc, dstv, ewv):
    cid = lax.axis_index("c")
    sid = lax.axis_index("s")
    wid = cid * NS + sid

    @pl.when(sid < 10)
    def _():
        pltpu.sync_copy(zero_hbm.at[pl.ds(sid * 1000, 1000)],
                        acc.at[pl.ds(sid * 1000, 1000)])

    pltpu.sync_copy(dst_hbm.at[pl.ds(wid * NBLK, NBLK)], dstv)
    pltpu.sync_copy(ew_hbm.at[pl.ds(wid * NBLK, NBLK)], ewv)
    plsc.subcore_barrier()

    def body(j, _):
        pltpu.sync_copy(ewv.at[j], acc.at[dstv.at[j]], add=True)
        return ()

    lax.fori_loop(0, NBLK, body, (), unroll=False)
    plsc.subcore_barrier()

    @pl.when(sid < 10)
    def _():
        pltpu.sync_copy(acc.at[pl.ds(sid * 1000, 1000)],
                        out_hbm.at[cid, pl.ds(sid * 1000, 1000)])


@functools.partial(
    pl.kernel,
    out_type=jax.ShapeDtypeStruct((NC * N_NODES_C, 16), jnp.float32),
    mesh=_MESH,
    scratch_types=[
        pltpu.VMEM_SHARED((N_NODES_C, 16), jnp.float32),
        pltpu.VMEM((NBLK, BLK), jnp.int32),
        pltpu.VMEM((NBLK, BLK), jnp.int32),
        pltpu.VMEM((NBLK, BLK), jnp.float32),
        pltpu.VMEM((BLK, 16), jnp.float32),
    ],
)
def _sc_aggregate(h_hbm, src_hbm, dst_hbm, ew_hbm, zero_hbm, out_hbm,
                  acc, srcv, dstv, ewv, rows):
    cid = lax.axis_index("c")
    sid = lax.axis_index("s")
    wid = cid * NS + sid

    pltpu.sync_copy(zero_hbm.at[pl.ds(sid * ROWS_PER_TILE, ROWS_PER_TILE)],
                    acc.at[pl.ds(sid * ROWS_PER_TILE, ROWS_PER_TILE)])
    pltpu.sync_copy(src_hbm.at[pl.ds(wid * NBLK, NBLK)], srcv)
    pltpu.sync_copy(dst_hbm.at[pl.ds(wid * NBLK, NBLK)], dstv)
    pltpu.sync_copy(ew_hbm.at[pl.ds(wid * NBLK, NBLK)], ewv)
    plsc.subcore_barrier()

    def body(j, _):
        pltpu.sync_copy(h_hbm.at[srcv.at[j]], rows)
        for i in range(BLK):
            rows[i, :] = rows[i, :] * ewv[j, i]
        pltpu.sync_copy(rows, acc.at[dstv.at[j]], add=True)
        return ()

    lax.fori_loop(0, NBLK, body, (), unroll=False)
    plsc.subcore_barrier()

    pltpu.sync_copy(
        acc.at[pl.ds(sid * ROWS_PER_TILE, ROWS_PER_TILE)],
        out_hbm.at[pl.ds(cid * N_NODES_C + sid * ROWS_PER_TILE,
                         ROWS_PER_TILE)])


def _tc1_body(x_ref, w_ref, dinv_ref, o_ref):
    h = jnp.dot(x_ref[...], w_ref[...], preferred_element_type=jnp.float32)
    o_ref[...] = h * dinv_ref[...]


def _tc_mid_body(agg_ref, hp_ref, b_ref, dinv_ref, o_ref):
    pre = (agg_ref[0] + agg_ref[1] + hp_ref[...]) * dinv_ref[...] + b_ref[...]
    o_ref[...] = jnp.maximum(pre, 0.0) * dinv_ref[...]


def _tc_fin_body(agg_ref, hp_ref, w_ref, b_ref, dinv_ref, o_ref):
    pre = (agg_ref[0] + agg_ref[1] + hp_ref[...]) * dinv_ref[...]
    o_ref[...] = jnp.dot(pre, w_ref[...],
                         preferred_element_type=jnp.float32) + b_ref[...]


def kernel(x, edge_index, edge_weight, W1, b1, W2, b2):
    src = edge_index[0].astype(jnp.int32).reshape(NW * NBLK, BLK)
    dst = edge_index[1].astype(jnp.int32).reshape(NW * NBLK, BLK)
    ew = edge_weight.reshape(NW * NBLK, BLK)
    zero1 = jnp.zeros((N_NODES_C,), jnp.float32)
    zero16 = jnp.zeros((N_NODES_C, 16), jnp.float32)

    degp = _sc_degree(dst, ew, zero1)
    dinv = lax.rsqrt(degp[0] + degp[1] + 1.0)[:, None]

    h1p = pl.pallas_call(
        _tc1_body,
        out_shape=jax.ShapeDtypeStruct((N_NODES_C, 16), jnp.float32),
    )(x, W1, dinv)

    agg1 = _sc_aggregate(h1p, src, dst, ew, zero16).reshape(2, N_NODES_C, 16)

    out1p = pl.pallas_call(
        _tc_mid_body,
        out_shape=jax.ShapeDtypeStruct((N_NODES_C, 16), jnp.float32),
    )(agg1, h1p, b1[None, :], dinv)

    agg2 = _sc_aggregate(out1p, src, dst, ew, zero16).reshape(2, N_NODES_C, 16)

    out = pl.pallas_call(
        _tc_fin_body,
        out_shape=jax.ShapeDtypeStruct((N_NODES_C, 64), jnp.float32),
    )(agg2, out1p, W2, b2[None, :], dinv)
    return out


# speedup vs baseline: 26.9405x; 26.9405x over previous
import functools

import jax
import jax.numpy as jnp
from jax import lax
from jax.experimental import pallas as pl
from jax.experimental.pallas import tpu as pltpu
from jax.experimental.pallas import tpu_sc as plsc

N_NODES_C = 10000
N_EDGES_C = 320000
NC = 2
NS = 16
NW = NC * NS
EDGES_PER_W = N_EDGES_C // NW
BLK = 80
NBLK = EDGES_PER_W // BLK
RPT = 640
N_PAD = NS * RPT

_MESH = plsc.VectorSubcoreMesh(core_axis_name="c", subcore_axis_name="s")


@functools.partial(
    pl.kernel,
    out_type=jax.ShapeDtypeStruct((NC * N_PAD,), jnp.float32),
    mesh=_MESH,
    scratch_types=[
        pltpu.VMEM_SHARED((N_PAD,), jnp.float32),
        pltpu.VMEM((NBLK, BLK), jnp.int32),
        pltpu.VMEM((NBLK, BLK), jnp.float32),
        pltpu.VMEM((RPT,), jnp.float32),
    ],
)
def _sc_degree(dst_hbm, ew_hbm, out_hbm, acc, dstv, ewv, zbuf):
    cid = lax.axis_index("c")
    sid = lax.axis_index("s")
    wid = cid * NS + sid

    z16 = jnp.zeros((16,), jnp.float32)
    for k in range(RPT // 16):
        zbuf[pl.ds(k * 16, 16)] = z16
    pltpu.sync_copy(zbuf, acc.at[pl.ds(sid * RPT, RPT)])

    pltpu.sync_copy(dst_hbm.at[wid], dstv)
    pltpu.sync_copy(ew_hbm.at[wid], ewv)
    plsc.subcore_barrier()

    def body(j, _):
        pltpu.sync_copy(ewv.at[j], acc.at[dstv.at[j]], add=True)
        return ()

    lax.fori_loop(0, NBLK, body, (), unroll=False)
    plsc.subcore_barrier()

    pltpu.sync_copy(acc.at[pl.ds(sid * RPT, RPT)],
                    out_hbm.at[pl.ds(cid * N_PAD + sid * RPT, RPT)])


@functools.partial(
    pl.kernel,
    out_type=jax.ShapeDtypeStruct((NC, NS, RPT, 16), jnp.float32),
    mesh=_MESH,
    compiler_params=pltpu.CompilerParams(use_tc_tiling_on_sc=False),
    scratch_types=[
        pltpu.VMEM_SHARED((N_PAD, 16), jnp.float32),
        pltpu.VMEM((NBLK, BLK), jnp.int32),
        pltpu.VMEM((NBLK, BLK), jnp.int32),
        pltpu.VMEM((NBLK, BLK), jnp.float32),
        pltpu.VMEM((BLK, 16), jnp.float32),
    ],
)
def _sc_aggregate(h_hbm, src_hbm, dst_hbm, ew_hbm, out_hbm,
                  acc, srcv, dstv, ewv, rows):
    cid = lax.axis_index("c")
    sid = lax.axis_index("s")
    wid = cid * NS + sid

    z16 = jnp.zeros((16,), jnp.float32)
    for i in range(BLK):
        rows[i, :] = z16
    for k in range(RPT // BLK):
        pltpu.sync_copy(rows, acc.at[pl.ds(sid * RPT + k * BLK, BLK)])

    pltpu.sync_copy(src_hbm.at[wid], srcv)
    pltpu.sync_copy(dst_hbm.at[wid], dstv)
    pltpu.sync_copy(ew_hbm.at[wid], ewv)
    plsc.subcore_barrier()

    def body(j, _):
        pltpu.sync_copy(h_hbm.at[srcv.at[j]], rows)
        for g in range(BLK // 16):
            w16 = ewv[j, pl.ds(g * 16, 16)]
            for i in range(16):
                r = g * 16 + i
                rows[r, :] = rows[r, :] * w16[i]
        pltpu.sync_copy(rows, acc.at[dstv.at[j]], add=True)
        return ()

    lax.fori_loop(0, NBLK, body, (), unroll=False)
    plsc.subcore_barrier()

    pltpu.sync_copy(acc.at[pl.ds(sid * RPT, RPT)], out_hbm.at[cid, sid])


def _tc1_body(x_ref, w_ref, dinv_ref, o_ref):
    h = jnp.dot(x_ref[...], w_ref[...], preferred_element_type=jnp.float32)
    o_ref[...] = h * dinv_ref[...]


def _tc_mid_body(agg_ref, hp_ref, b_ref, dinv_ref, o_ref):
    pre = (agg_ref[0] + agg_ref[1] + hp_ref[...]) * dinv_ref[...] + b_ref[...]
    o_ref[...] = jnp.maximum(pre, 0.0) * dinv_ref[...]


def _tc_fin_body(agg_ref, hp_ref, w_ref, b_ref, dinv_ref, o_ref):
    pre = (agg_ref[0] + agg_ref[1] + hp_ref[...]) * dinv_ref[...]
    o_ref[...] = jnp.dot(pre, w_ref[...],
                         preferred_element_type=jnp.float32) + b_ref[...]


def kernel(x, edge_index, edge_weight, W1, b1, W2, b2):
    src = edge_index[0].astype(jnp.int32).reshape(NW, NBLK, BLK)
    dst = edge_index[1].astype(jnp.int32).reshape(NW, NBLK, BLK)
    ew = edge_weight.reshape(NW, NBLK, BLK)

    degp = _sc_degree(dst, ew).reshape(NC, N_PAD)
    dinv = lax.rsqrt(degp[0, :N_NODES_C] + degp[1, :N_NODES_C] + 1.0)[:, None]

    h1p = pl.pallas_call(
        _tc1_body,
        out_shape=jax.ShapeDtypeStruct((N_NODES_C, 16), jnp.float32),
    )(x, W1, dinv)

    agg1 = _sc_aggregate(h1p, src, dst, ew).reshape(
        NC, N_PAD, 16)[:, :N_NODES_C]

    out1p = pl.pallas_call(
        _tc_mid_body,
        out_shape=jax.ShapeDtypeStruct((N_NODES_C, 16), jnp.float32),
    )(agg1, h1p, b1[None, :], dinv)

    agg2 = _sc_aggregate(out1p, src, dst, ew).reshape(
        NC, N_PAD, 16)[:, :N_NODES_C]

    out = pl.pallas_call(
        _tc_fin_body,
        out_shape=jax.ShapeDtypeStruct((N_NODES_C, 64), jnp.float32),
    )(agg2, out1p, W2, b2[None, :], dinv)
    return out
